# baseline (speedup 1.0000x reference)
import jax
import jax.numpy as jnp
from jax.experimental import pallas as pl

_P = 0.1
_CB = 4


def _body(rand_ref, mask_ref, x_ref, out_ref, xout_ref):
    keep = (rand_ref[...] >= _P).astype(jnp.float32)
    out_ref[...] = mask_ref[...] * keep[None]
    xout_ref[...] = x_ref[...]


def kernel(x, mask, rand_t):
    C, T, V = mask.shape
    mask_t = jnp.transpose(mask, (0, 2, 1))
    x_t = jnp.transpose(x, (0, 2, 1))

    blk = pl.BlockSpec((_CB, V, T), lambda i: (i, 0, 0))
    out_t, xout_t = pl.pallas_call(
        _body,
        grid=(C // _CB,),
        in_specs=[
            pl.BlockSpec((1, T), lambda i: (0, 0)),
            blk,
            blk,
        ],
        out_specs=[blk, blk],
        out_shape=[
            jax.ShapeDtypeStruct((C, V, T), jnp.float32),
            jax.ShapeDtypeStruct((C, V, T), jnp.float32),
        ],
    )(rand_t.reshape(1, T), mask_t, x_t)
    return (jnp.transpose(xout_t, (0, 2, 1)), jnp.transpose(out_t, (0, 2, 1)))


# speedup vs baseline: 1.0128x; 1.0128x over previous
import jax
import jax.numpy as jnp
from jax.experimental import pallas as pl

_P = 0.1
_CB = 8


def _body(rand_ref, mask_ref, x_ref, out_ref, xout_ref):
    keep = (rand_ref[...] >= _P).astype(jnp.float32)
    out_ref[...] = mask_ref[...] * keep[None]
    xout_ref[...] = x_ref[...]


def kernel(x, mask, rand_t):
    C, T, V = mask.shape
    mask_t = jnp.transpose(mask, (0, 2, 1))
    x_t = jnp.transpose(x, (0, 2, 1))

    blk = pl.BlockSpec((_CB, V, T), lambda i: (i, 0, 0))
    out_t, xout_t = pl.pallas_call(
        _body,
        grid=(C // _CB,),
        in_specs=[
            pl.BlockSpec((1, T), lambda i: (0, 0)),
            blk,
            blk,
        ],
        out_specs=[blk, blk],
        out_shape=[
            jax.ShapeDtypeStruct((C, V, T), jnp.float32),
            jax.ShapeDtypeStruct((C, V, T), jnp.float32),
        ],
    )(rand_t.reshape(1, T), mask_t, x_t)
    return (jnp.transpose(xout_t, (0, 2, 1)), jnp.transpose(out_t, (0, 2, 1)))
